# baseline (speedup 1.0000x reference)
import jax
import jax.numpy as jnp
from jax import lax
from jax.experimental import pallas as pl
from jax.experimental.pallas import tpu as pltpu
from jax.experimental.pallas import tpu_sc as plsc

N = 100000
H = 128
S = 512
NC = 2
NS = 16
NW = NC * NS
CHUNK = 112
NCH = 16
BASE = NCH * CHUNK
M_SC = NW * BASE
N_TC = N - M_SC
BM = 1376
NB = N_TC // BM
ROWS_PER_TILE = S // NS


def _pool_body(h_hbm, b3_hbm, z128_hbm, z512_hbm, part_out, cnt_out,
               acc_sh, rows0_v, rows1_v, idx2_v, cnt_v, sem0, sem1):
    c = lax.axis_index("c")
    s = lax.axis_index("s")
    wid = c * NS + s
    base = N_TC + wid * BASE

    pltpu.sync_copy(z128_hbm.at[pl.ds(s * ROWS_PER_TILE, ROWS_PER_TILE)],
                    acc_sh.at[pl.ds(s * ROWS_PER_TILE, ROWS_PER_TILE)])
    pltpu.sync_copy(z512_hbm, cnt_v)
    pltpu.sync_copy(b3_hbm.at[wid], idx2_v)

    plsc.subcore_barrier()

    ones16 = jnp.full((16,), 1.0, jnp.float32)

    def _off(i):
        return base + i * CHUNK

    def _counts(i):
        for k in range(CHUNK // 16):
            plsc.addupdate_scatter(cnt_v, [idx2_v[i, pl.ds(16 * k, 16)]],
                                   ones16)

    pltpu.async_copy(h_hbm.at[pl.ds(base, CHUNK)], rows0_v, sem0)

    def _pair(j, _):
        i0 = 2 * j
        i1 = 2 * j + 1
        pltpu.async_copy(h_hbm.at[pl.ds(_off(i1), CHUNK)], rows1_v, sem1)
        pltpu.make_async_copy(h_hbm.at[pl.ds(_off(i0), CHUNK)], rows0_v,
                              sem0).wait()
        pltpu.sync_copy(rows0_v, acc_sh.at[idx2_v.at[i0]], add=True)
        _counts(i0)
        off2 = jnp.minimum(_off(i0 + 2), N - CHUNK)
        pltpu.async_copy(h_hbm.at[pl.ds(off2, CHUNK)], rows0_v, sem0)
        pltpu.make_async_copy(h_hbm.at[pl.ds(_off(i1), CHUNK)], rows1_v,
                              sem1).wait()
        pltpu.sync_copy(rows1_v, acc_sh.at[idx2_v.at[i1]], add=True)
        _counts(i1)
        return _

    lax.fori_loop(0, NCH // 2, _pair, 0)

    pltpu.make_async_copy(h_hbm.at[pl.ds(0, CHUNK)], rows0_v, sem0).wait()

    plsc.subcore_barrier()

    r0 = s * ROWS_PER_TILE
    pltpu.sync_copy(acc_sh.at[pl.ds(r0, ROWS_PER_TILE)],
                    part_out.at[c, pl.ds(r0, ROWS_PER_TILE)])
    pltpu.sync_copy(cnt_v, cnt_out.at[c, s])


@jax.jit
def _sc_pool(h, b3d, z128, z512):
    mesh = plsc.VectorSubcoreMesh(core_axis_name="c", subcore_axis_name="s")
    f = pl.kernel(
        _pool_body,
        out_type=(
            jax.ShapeDtypeStruct((NC, S, H), jnp.float32),
            jax.ShapeDtypeStruct((NC, NS, S), jnp.float32),
        ),
        mesh=mesh,
        compiler_params=pltpu.CompilerParams(needs_layout_passes=False),
        scratch_types=[
            pltpu.VMEM_SHARED((S, H), jnp.float32),
            pltpu.VMEM((CHUNK, H), jnp.float32),
            pltpu.VMEM((CHUNK, H), jnp.float32),
            pltpu.VMEM((NCH, CHUNK), jnp.int32),
            pltpu.VMEM((S,), jnp.float32),
            pltpu.SemaphoreType.DMA,
            pltpu.SemaphoreType.DMA,
        ],
    )
    return f(h, b3d, z128, z512)


def _tc_body(ids_ref, h_ref, p_ref, c_ref):
    i = pl.program_id(0)
    seg = lax.broadcasted_iota(jnp.int32, (S, BM), 0)
    onehot = (ids_ref[0] == seg).astype(jnp.float32)
    ps = jnp.dot(onehot, h_ref[...], preferred_element_type=jnp.float32,
                 precision=lax.Precision.HIGHEST)
    cs = jnp.sum(onehot, axis=1, keepdims=True)

    @pl.when(i == 0)
    def _():
        p_ref[...] = jnp.zeros_like(p_ref)
        c_ref[...] = jnp.zeros_like(c_ref)

    p_ref[...] += ps
    c_ref[...] += cs


@jax.jit
def _tc_pool(ids_tc, h):
    return pl.pallas_call(
        _tc_body,
        grid=(NB,),
        in_specs=[
            pl.BlockSpec((1, 1, BM), lambda i: (i, 0, 0)),
            pl.BlockSpec((BM, H), lambda i: (i, 0)),
        ],
        out_specs=[
            pl.BlockSpec((S, H), lambda i: (0, 0)),
            pl.BlockSpec((S, 1), lambda i: (0, 0)),
        ],
        out_shape=[
            jax.ShapeDtypeStruct((S, H), jnp.float32),
            jax.ShapeDtypeStruct((S, 1), jnp.float32),
        ],
    )(ids_tc, h)


def _combine_body(p_ref, c_ref, ptc_ref, ctc_ref, o_ref):
    p = p_ref[0] + p_ref[1] + ptc_ref[...]
    cnt = jnp.sum(c_ref[...], axis=(0, 1)).reshape(S, 1) + ctc_ref[...]
    cnt = jnp.maximum(cnt, 1.0)
    o_ref[...] = p / cnt


@jax.jit
def _combine(part, cnt, ptc, ctc):
    return pl.pallas_call(
        _combine_body,
        out_shape=jax.ShapeDtypeStruct((S, H), jnp.float32),
    )(part, cnt, ptc, ctc)


def kernel(h, batch):
    b32 = batch.astype(jnp.int32)
    b3d = b32[N_TC:].reshape(NW, NCH, CHUNK)
    ids_tc = b32[:N_TC].reshape(NB, 1, BM)
    z128 = jnp.zeros((S, H), jnp.float32)
    z512 = jnp.zeros((S,), jnp.float32)
    part, cnt = _sc_pool(h, b3d, z128, z512)
    ptc, ctc = _tc_pool(ids_tc, h)
    return _combine(part, cnt, ptc, ctc)


# speedup vs baseline: 1.9320x; 1.9320x over previous
import jax
import jax.numpy as jnp
from jax import lax
from jax.experimental import pallas as pl
from jax.experimental.pallas import tpu as pltpu
from jax.experimental.pallas import tpu_sc as plsc

N = 100000
H = 128
S = 512
NC = 2
NS = 16
NW = NC * NS
CHUNK = 112
NCH = 16
BASE = NCH * CHUNK
M_SC = NW * BASE
N_TC = N - M_SC
BM = 1376
NB = N_TC // BM
ROWS_PER_TILE = S // NS


def _pool_body(h_hbm, b3_hbm, z128_hbm, z512_hbm, part_out, cnt_out,
               acc_sh, rows0_v, rows1_v, idx2_v, cnt_v, sem0, sem1):
    c = lax.axis_index("c")
    s = lax.axis_index("s")
    wid = c * NS + s
    base = N_TC + wid * BASE

    pltpu.sync_copy(z128_hbm.at[pl.ds(s * ROWS_PER_TILE, ROWS_PER_TILE)],
                    acc_sh.at[pl.ds(s * ROWS_PER_TILE, ROWS_PER_TILE)])
    pltpu.sync_copy(z512_hbm, cnt_v)
    pltpu.sync_copy(b3_hbm.at[wid], idx2_v)

    plsc.subcore_barrier()

    ones16 = jnp.full((16,), 1.0, jnp.float32)

    def _off(i):
        return base + i * CHUNK

    def _counts(i):
        for k in range(CHUNK // 16):
            plsc.addupdate_scatter(cnt_v, [idx2_v[i, pl.ds(16 * k, 16)]],
                                   ones16)

    pltpu.async_copy(h_hbm.at[pl.ds(base, CHUNK)], rows0_v, sem0)

    def _pair(j, _):
        i0 = 2 * j
        i1 = 2 * j + 1
        pltpu.async_copy(h_hbm.at[pl.ds(_off(i1), CHUNK)], rows1_v, sem1)
        pltpu.make_async_copy(h_hbm.at[pl.ds(_off(i0), CHUNK)], rows0_v,
                              sem0).wait()
        pltpu.sync_copy(rows0_v, acc_sh.at[idx2_v.at[i0]], add=True)
        _counts(i0)
        off2 = jnp.minimum(_off(i0 + 2), N - CHUNK)
        pltpu.async_copy(h_hbm.at[pl.ds(off2, CHUNK)], rows0_v, sem0)
        pltpu.make_async_copy(h_hbm.at[pl.ds(_off(i1), CHUNK)], rows1_v,
                              sem1).wait()
        pltpu.sync_copy(rows1_v, acc_sh.at[idx2_v.at[i1]], add=True)
        _counts(i1)
        return _

    lax.fori_loop(0, NCH // 2, _pair, 0)

    pltpu.make_async_copy(h_hbm.at[pl.ds(0, CHUNK)], rows0_v, sem0).wait()

    plsc.subcore_barrier()

    r0 = s * ROWS_PER_TILE
    pltpu.sync_copy(acc_sh.at[pl.ds(r0, ROWS_PER_TILE)],
                    part_out.at[c, pl.ds(r0, ROWS_PER_TILE)])
    pltpu.sync_copy(cnt_v, cnt_out.at[c, s])


@jax.jit
def _sc_pool(h, b3d, z128, z512):
    mesh = plsc.VectorSubcoreMesh(core_axis_name="c", subcore_axis_name="s")
    f = pl.kernel(
        _pool_body,
        out_type=(
            jax.ShapeDtypeStruct((NC, S, H), jnp.float32),
            jax.ShapeDtypeStruct((NC, NS, S), jnp.float32),
        ),
        mesh=mesh,
        compiler_params=pltpu.CompilerParams(needs_layout_passes=False),
        scratch_types=[
            pltpu.VMEM_SHARED((S, H), jnp.float32),
            pltpu.VMEM((CHUNK, H), jnp.float32),
            pltpu.VMEM((CHUNK, H), jnp.float32),
            pltpu.VMEM((NCH, CHUNK), jnp.int32),
            pltpu.VMEM((S,), jnp.float32),
            pltpu.SemaphoreType.DMA,
            pltpu.SemaphoreType.DMA,
        ],
    )
    return f(h, b3d, z128, z512)


def _tc_body(ids_ref, h_ref, p_ref, c_ref):
    i = pl.program_id(0)
    seg = lax.broadcasted_iota(jnp.int32, (S, BM), 0)
    onehot = (ids_ref[0] == seg).astype(jnp.bfloat16)
    hb = h_ref[...].astype(jnp.bfloat16)
    ps = jnp.dot(onehot, hb, preferred_element_type=jnp.float32)
    cs = jnp.sum(onehot.astype(jnp.float32), axis=1, keepdims=True)

    @pl.when(i == 0)
    def _():
        p_ref[...] = jnp.zeros_like(p_ref)
        c_ref[...] = jnp.zeros_like(c_ref)

    p_ref[...] += ps
    c_ref[...] += cs


@jax.jit
def _tc_pool(ids_tc, h):
    return pl.pallas_call(
        _tc_body,
        grid=(NB,),
        in_specs=[
            pl.BlockSpec((1, 1, BM), lambda i: (i, 0, 0)),
            pl.BlockSpec((BM, H), lambda i: (i, 0)),
        ],
        out_specs=[
            pl.BlockSpec((S, H), lambda i: (0, 0)),
            pl.BlockSpec((S, 1), lambda i: (0, 0)),
        ],
        out_shape=[
            jax.ShapeDtypeStruct((S, H), jnp.float32),
            jax.ShapeDtypeStruct((S, 1), jnp.float32),
        ],
    )(ids_tc, h)


def _combine_body(p_ref, c_ref, ptc_ref, ctc_ref, o_ref):
    p = p_ref[0] + p_ref[1] + ptc_ref[...]
    cnt = jnp.sum(c_ref[...], axis=(0, 1)).reshape(S, 1) + ctc_ref[...]
    cnt = jnp.maximum(cnt, 1.0)
    o_ref[...] = p / cnt


@jax.jit
def _combine(part, cnt, ptc, ctc):
    return pl.pallas_call(
        _combine_body,
        out_shape=jax.ShapeDtypeStruct((S, H), jnp.float32),
    )(part, cnt, ptc, ctc)


def kernel(h, batch):
    b32 = batch.astype(jnp.int32)
    b3d = b32[N_TC:].reshape(NW, NCH, CHUNK)
    ids_tc = b32[:N_TC].reshape(NB, 1, BM)
    z128 = jnp.zeros((S, H), jnp.float32)
    z512 = jnp.zeros((S,), jnp.float32)
    part, cnt = _sc_pool(h, b3d, z128, z512)
    ptc, ctc = _tc_pool(ids_tc, h)
    return _combine(part, cnt, ptc, ctc)


# speedup vs baseline: 2.1616x; 1.1188x over previous
import jax
import jax.numpy as jnp
from jax import lax
from jax.experimental import pallas as pl
from jax.experimental.pallas import tpu as pltpu
from jax.experimental.pallas import tpu_sc as plsc

N = 100000
H = 128
S = 512
NC = 2
NS = 16
NW = NC * NS
CHUNK = 112
NCH = 17
BASE = NCH * CHUNK
M_SC = NW * BASE
N_TC = N - M_SC
BM = 3552
NB = N_TC // BM
ROWS_PER_TILE = S // NS


def _pool_body(h_hbm, b3_hbm, z128_hbm, z512_hbm, part_out, cnt_out,
               acc_sh, rows0_v, rows1_v, idx2_v, cnt_v, sem0, sem1):
    c = lax.axis_index("c")
    s = lax.axis_index("s")
    wid = c * NS + s
    base = N_TC + wid * BASE

    pltpu.sync_copy(z128_hbm.at[pl.ds(s * ROWS_PER_TILE, ROWS_PER_TILE)],
                    acc_sh.at[pl.ds(s * ROWS_PER_TILE, ROWS_PER_TILE)])
    pltpu.sync_copy(z512_hbm, cnt_v)
    pltpu.sync_copy(b3_hbm.at[wid], idx2_v)

    plsc.subcore_barrier()

    ones16 = jnp.full((16,), 1.0, jnp.float32)

    def _off(i):
        return base + i * CHUNK

    def _counts(i):
        for k in range(CHUNK // 16):
            plsc.addupdate_scatter(cnt_v, [idx2_v[i, pl.ds(16 * k, 16)]],
                                   ones16)

    pltpu.async_copy(h_hbm.at[pl.ds(base, CHUNK)], rows0_v, sem0)

    def _pair(j, _):
        i0 = 2 * j
        i1 = 2 * j + 1
        pltpu.async_copy(h_hbm.at[pl.ds(_off(i1), CHUNK)], rows1_v, sem1)
        pltpu.make_async_copy(h_hbm.at[pl.ds(_off(i0), CHUNK)], rows0_v,
                              sem0).wait()
        pltpu.sync_copy(rows0_v, acc_sh.at[idx2_v.at[i0]], add=True)
        _counts(i0)
        off2 = jnp.minimum(_off(i0 + 2), N - CHUNK)
        pltpu.async_copy(h_hbm.at[pl.ds(off2, CHUNK)], rows0_v, sem0)
        pltpu.make_async_copy(h_hbm.at[pl.ds(_off(i1), CHUNK)], rows1_v,
                              sem1).wait()
        pltpu.sync_copy(rows1_v, acc_sh.at[idx2_v.at[i1]], add=True)
        _counts(i1)
        return _

    lax.fori_loop(0, NCH // 2, _pair, 0)

    pltpu.make_async_copy(h_hbm.at[pl.ds(_off(NCH - 1), CHUNK)], rows0_v,
                          sem0).wait()
    pltpu.sync_copy(rows0_v, acc_sh.at[idx2_v.at[NCH - 1]], add=True)
    _counts(NCH - 1)

    plsc.subcore_barrier()

    r0 = s * ROWS_PER_TILE
    pltpu.sync_copy(acc_sh.at[pl.ds(r0, ROWS_PER_TILE)],
                    part_out.at[c, pl.ds(r0, ROWS_PER_TILE)])
    pltpu.sync_copy(cnt_v, cnt_out.at[c, s])


@jax.jit
def _sc_pool(h, b3d, z128, z512):
    mesh = plsc.VectorSubcoreMesh(core_axis_name="c", subcore_axis_name="s")
    f = pl.kernel(
        _pool_body,
        out_type=(
            jax.ShapeDtypeStruct((NC, S, H), jnp.float32),
            jax.ShapeDtypeStruct((NC, NS, S), jnp.float32),
        ),
        mesh=mesh,
        compiler_params=pltpu.CompilerParams(needs_layout_passes=False),
        scratch_types=[
            pltpu.VMEM_SHARED((S, H), jnp.float32),
            pltpu.VMEM((CHUNK, H), jnp.float32),
            pltpu.VMEM((CHUNK, H), jnp.float32),
            pltpu.VMEM((NCH, CHUNK), jnp.int32),
            pltpu.VMEM((S,), jnp.float32),
            pltpu.SemaphoreType.DMA,
            pltpu.SemaphoreType.DMA,
        ],
    )
    return f(h, b3d, z128, z512)


def _tc_body(ids_ref, h_ref, p_ref, c_ref):
    i = pl.program_id(0)
    seg = lax.broadcasted_iota(jnp.int32, (S, BM), 0)
    onehot = (ids_ref[0] == seg).astype(jnp.bfloat16)
    hb = h_ref[...].astype(jnp.bfloat16)
    ps = jnp.dot(onehot, hb, preferred_element_type=jnp.float32)
    cs = jnp.sum(onehot.astype(jnp.float32), axis=1, keepdims=True)

    @pl.when(i == 0)
    def _():
        p_ref[...] = jnp.zeros_like(p_ref)
        c_ref[...] = jnp.zeros_like(c_ref)

    p_ref[...] += ps
    c_ref[...] += cs


@jax.jit
def _tc_pool(ids_tc, h):
    return pl.pallas_call(
        _tc_body,
        grid=(NB,),
        in_specs=[
            pl.BlockSpec((1, 1, BM), lambda i: (i, 0, 0)),
            pl.BlockSpec((BM, H), lambda i: (i, 0)),
        ],
        out_specs=[
            pl.BlockSpec((S, H), lambda i: (0, 0)),
            pl.BlockSpec((S, 1), lambda i: (0, 0)),
        ],
        out_shape=[
            jax.ShapeDtypeStruct((S, H), jnp.float32),
            jax.ShapeDtypeStruct((S, 1), jnp.float32),
        ],
    )(ids_tc, h)


def _combine_body(p_ref, c_ref, ptc_ref, ctc_ref, o_ref):
    p = p_ref[0] + p_ref[1] + ptc_ref[...]
    cnt = jnp.sum(c_ref[...], axis=(0, 1)).reshape(S, 1) + ctc_ref[...]
    cnt = jnp.maximum(cnt, 1.0)
    o_ref[...] = p / cnt


@jax.jit
def _combine(part, cnt, ptc, ctc):
    return pl.pallas_call(
        _combine_body,
        out_shape=jax.ShapeDtypeStruct((S, H), jnp.float32),
    )(part, cnt, ptc, ctc)


def kernel(h, batch):
    b32 = batch.astype(jnp.int32)
    b3d = b32[N_TC:].reshape(NW, NCH, CHUNK)
    ids_tc = b32[:N_TC].reshape(NB, 1, BM)
    z128 = jnp.zeros((S, H), jnp.float32)
    z512 = jnp.zeros((S,), jnp.float32)
    part, cnt = _sc_pool(h, b3d, z128, z512)
    ptc, ctc = _tc_pool(ids_tc, h)
    return _combine(part, cnt, ptc, ctc)


# speedup vs baseline: 2.2546x; 1.0430x over previous
import jax
import jax.numpy as jnp
from jax import lax
from jax.experimental import pallas as pl
from jax.experimental.pallas import tpu as pltpu
from jax.experimental.pallas import tpu_sc as plsc

N = 100000
H = 128
S = 512
NC = 2
NS = 16
NW = NC * NS
CHUNK = 112
NCH = 15
BASE = NCH * CHUNK
M_SC = NW * BASE
N_TC = N - M_SC
BM = 4624
NB = N_TC // BM
ROWS_PER_TILE = S // NS


def _pool_body(h_hbm, b3_hbm, z128_hbm, z512_hbm, part_out, cnt_out,
               acc_sh, rows0_v, rows1_v, idx2_v, cnt_v, sem0, sem1):
    c = lax.axis_index("c")
    s = lax.axis_index("s")
    wid = c * NS + s
    base = N_TC + wid * BASE

    pltpu.sync_copy(z128_hbm.at[pl.ds(s * ROWS_PER_TILE, ROWS_PER_TILE)],
                    acc_sh.at[pl.ds(s * ROWS_PER_TILE, ROWS_PER_TILE)])
    pltpu.sync_copy(z512_hbm, cnt_v)
    pltpu.sync_copy(b3_hbm.at[wid], idx2_v)

    plsc.subcore_barrier()

    ones16 = jnp.full((16,), 1.0, jnp.float32)

    def _off(i):
        return base + i * CHUNK

    def _counts(i):
        for k in range(CHUNK // 16):
            plsc.addupdate_scatter(cnt_v, [idx2_v[i, pl.ds(16 * k, 16)]],
                                   ones16)

    pltpu.async_copy(h_hbm.at[pl.ds(base, CHUNK)], rows0_v, sem0)

    def _pair(j, _):
        i0 = 2 * j
        i1 = 2 * j + 1
        pltpu.async_copy(h_hbm.at[pl.ds(_off(i1), CHUNK)], rows1_v, sem1)
        pltpu.make_async_copy(h_hbm.at[pl.ds(_off(i0), CHUNK)], rows0_v,
                              sem0).wait()
        pltpu.sync_copy(rows0_v, acc_sh.at[idx2_v.at[i0]], add=True)
        _counts(i0)
        off2 = jnp.minimum(_off(i0 + 2), N - CHUNK)
        pltpu.async_copy(h_hbm.at[pl.ds(off2, CHUNK)], rows0_v, sem0)
        pltpu.make_async_copy(h_hbm.at[pl.ds(_off(i1), CHUNK)], rows1_v,
                              sem1).wait()
        pltpu.sync_copy(rows1_v, acc_sh.at[idx2_v.at[i1]], add=True)
        _counts(i1)
        return _

    lax.fori_loop(0, NCH // 2, _pair, 0)

    pltpu.make_async_copy(h_hbm.at[pl.ds(_off(NCH - 1), CHUNK)], rows0_v,
                          sem0).wait()
    pltpu.sync_copy(rows0_v, acc_sh.at[idx2_v.at[NCH - 1]], add=True)
    _counts(NCH - 1)

    plsc.subcore_barrier()

    r0 = s * ROWS_PER_TILE
    pltpu.sync_copy(acc_sh.at[pl.ds(r0, ROWS_PER_TILE)],
                    part_out.at[c, pl.ds(r0, ROWS_PER_TILE)])
    pltpu.sync_copy(cnt_v, cnt_out.at[c, s])


@jax.jit
def _sc_pool(h, b3d, z128, z512):
    mesh = plsc.VectorSubcoreMesh(core_axis_name="c", subcore_axis_name="s")
    f = pl.kernel(
        _pool_body,
        out_type=(
            jax.ShapeDtypeStruct((NC, S, H), jnp.float32),
            jax.ShapeDtypeStruct((NC, NS, S), jnp.float32),
        ),
        mesh=mesh,
        compiler_params=pltpu.CompilerParams(needs_layout_passes=False),
        scratch_types=[
            pltpu.VMEM_SHARED((S, H), jnp.float32),
            pltpu.VMEM((CHUNK, H), jnp.float32),
            pltpu.VMEM((CHUNK, H), jnp.float32),
            pltpu.VMEM((NCH, CHUNK), jnp.int32),
            pltpu.VMEM((S,), jnp.float32),
            pltpu.SemaphoreType.DMA,
            pltpu.SemaphoreType.DMA,
        ],
    )
    return f(h, b3d, z128, z512)


def _tc_body(ids_ref, h_ref, p_ref, c_ref):
    i = pl.program_id(0)
    seg = lax.broadcasted_iota(jnp.int32, (S, BM), 0)
    onehot = (ids_ref[0] == seg).astype(jnp.bfloat16)
    hb = h_ref[...].astype(jnp.bfloat16)
    ps = jnp.dot(onehot, hb, preferred_element_type=jnp.float32)
    cs = jnp.sum(onehot.astype(jnp.float32), axis=1, keepdims=True)

    @pl.when(i == 0)
    def _():
        p_ref[...] = jnp.zeros_like(p_ref)
        c_ref[...] = jnp.zeros_like(c_ref)

    p_ref[...] += ps
    c_ref[...] += cs


@jax.jit
def _tc_pool(ids_tc, h):
    return pl.pallas_call(
        _tc_body,
        grid=(NB,),
        in_specs=[
            pl.BlockSpec((1, 1, BM), lambda i: (i, 0, 0)),
            pl.BlockSpec((BM, H), lambda i: (i, 0)),
        ],
        out_specs=[
            pl.BlockSpec((S, H), lambda i: (0, 0)),
            pl.BlockSpec((S, 1), lambda i: (0, 0)),
        ],
        out_shape=[
            jax.ShapeDtypeStruct((S, H), jnp.float32),
            jax.ShapeDtypeStruct((S, 1), jnp.float32),
        ],
    )(ids_tc, h)


def _combine_body(p_ref, c_ref, ptc_ref, ctc_ref, o_ref):
    p = p_ref[0] + p_ref[1] + ptc_ref[...]
    cnt = jnp.sum(c_ref[...], axis=(0, 1)).reshape(S, 1) + ctc_ref[...]
    cnt = jnp.maximum(cnt, 1.0)
    o_ref[...] = p / cnt


@jax.jit
def _combine(part, cnt, ptc, ctc):
    return pl.pallas_call(
        _combine_body,
        out_shape=jax.ShapeDtypeStruct((S, H), jnp.float32),
    )(part, cnt, ptc, ctc)


def kernel(h, batch):
    b32 = batch.astype(jnp.int32)
    b3d = b32[N_TC:].reshape(NW, NCH, CHUNK)
    ids_tc = b32[:N_TC].reshape(NB, 1, BM)
    z128 = jnp.zeros((S, H), jnp.float32)
    z512 = jnp.zeros((S,), jnp.float32)
    part, cnt = _sc_pool(h, b3d, z128, z512)
    ptc, ctc = _tc_pool(ids_tc, h)
    return _combine(part, cnt, ptc, ctc)
